# baseline (speedup 1.0000x reference)
import functools

import jax
import jax.numpy as jnp
from jax import lax
from jax.experimental import pallas as pl
from jax.experimental.pallas import tpu as pltpu
from jax.experimental.pallas import tpu_sc as plsc

D_MODEL = 64
CUT0, CUT1, CUT2 = 20000, 200000, 1000000

NC, NS, L = 2, 16, 16
NW = NC * NS
G = 256


def _make_sc_lookup(n):
    per_w = n // NW
    assert n % NW == 0 and per_w % L == 0
    steps = per_w // L
    cap = per_w + G + L

    mesh = plsc.VectorSubcoreMesh(core_axis_name="c", subcore_axis_name="s")

    @functools.partial(
        pl.kernel,
        mesh=mesh,
        compiler_params=pltpu.CompilerParams(
            use_tc_tiling_on_sc=False, needs_layout_passes=False),
        out_type=jax.ShapeDtypeStruct((n, D_MODEL), jnp.float32),
        scratch_types=[
            pltpu.VMEM((per_w,), jnp.int32),
            pltpu.VMEM((cap,), jnp.int32),
            pltpu.VMEM((cap,), jnp.int32),
            pltpu.VMEM((cap,), jnp.int32),
            pltpu.VMEM((cap,), jnp.int32),
            pltpu.VMEM((cap,), jnp.int32),
            pltpu.VMEM((cap,), jnp.int32),
            pltpu.VMEM((G,), jnp.int32),
            pltpu.VMEM((G,), jnp.int32),
            pltpu.VMEM((G, D_MODEL), jnp.float32),
            pltpu.VMEM((G, D_MODEL), jnp.float32),
            pltpu.SemaphoreType.DMA,
            pltpu.SemaphoreType.DMA,
        ],
    )
    def sc_lookup(x_hbm, head_hbm, tail0_hbm, tail1_hbm, out_hbm,
                  idx_v, il0, pl0_, il1, pl1_, il2, pl2_,
                  ps0, ps1, rows0, rows1, sem_g, sem_s):
        wid = lax.axis_index("s") * NC + lax.axis_index("c")
        base = wid * per_w
        pltpu.sync_copy(x_hbm.at[pl.ds(base, per_w)], idx_v)

        lanes = lax.iota(jnp.int32, L)

        dump = jnp.int32(cap - 1)

        def compact(g, offs):
            o0, o1, o2 = offs
            v = idx_v[pl.ds(g * L, L)]
            pos = base + g * L + lanes
            m0 = v < CUT0
            m2 = v >= CUT1
            m1 = jnp.logical_not(m0) & jnp.logical_not(m2)
            new_offs = []
            for m, il, plref, adj, o in (
                (m0, il0, pl0_, v, o0),
                (m1, il1, pl1_, v - CUT0, o1),
                (m2, il2, pl2_, v - CUT1, o2),
            ):
                mi = m.astype(jnp.int32)
                pc = plsc.cumsum(mi)
                dest = jnp.where(m, o + pc - 1, dump)
                plsc.store_scatter(il, [dest], adj)
                plsc.store_scatter(plref, [dest], pos)
                new_offs.append(o + pc[L - 1])
            return tuple(new_offs)

        zero = jnp.int32(0)
        c0, c1, c2 = lax.fori_loop(0, steps, compact, (zero, zero, zero))

        for table, il, plist, cnt in (
            (head_hbm, il0, pl0_, c0),
            (tail0_hbm, il1, pl1_, c1),
            (tail1_hbm, il2, pl2_, c2),
        ):
            @pl.when(cnt % G != 0)
            def _pad(il=il, plist=plist, cnt=cnt):
                last = jnp.full((L,), cnt - 1, jnp.int32)
                lastv = plsc.load_gather(il, [last])
                lastp = plsc.load_gather(plist, [last])
                for k in range(G // L):
                    il[pl.ds(cnt + k * L, L)] = lastv
                    plist[pl.ds(cnt + k * L, L)] = lastp

            nsub = (cnt + (G - 1)) // G
            rows_bufs = (rows0, rows1)
            ps_bufs = (ps0, ps1)

            def issue_gather(j, parity, table=table, il=il):
                for p in range(2):
                    @pl.when(parity == p)
                    def _():
                        pltpu.async_copy(
                            table.at[il.at[pl.ds(j * G, G)]], rows_bufs[p],
                            sem_g)

            def wait_gather(table=table):
                pltpu.make_async_copy(
                    table.at[il0.at[pl.ds(0, G)]], rows0, sem_g).wait()

            def wait_scatter():
                pltpu.make_async_copy(rows0, out_hbm.at[ps0], sem_s).wait()

            @pl.when(nsub > 0)
            def _(table=table, il=il, plist=plist, nsub=nsub):
                issue_gather(0, 0)

                def gbody(j, _, table=table, il=il, plist=plist, nsub=nsub):
                    parity = j % 2

                    @pl.when(j >= 1)
                    def _():
                        wait_scatter()

                    wait_gather()
                    for p in range(2):
                        @pl.when(parity == p)
                        def _(p=p):
                            for k in range(G // L):
                                ps_bufs[p][pl.ds(k * L, L)] = (
                                    plist[pl.ds(j * G + k * L, L)])
                            pltpu.async_copy(
                                rows_bufs[p], out_hbm.at[ps_bufs[p]], sem_s)

                    @pl.when(j + 1 < nsub)
                    def _():
                        issue_gather(j + 1, (j + 1) % 2)
                    return 0

                lax.fori_loop(0, nsub, gbody, 0)
                wait_scatter()

    return sc_lookup


def kernel(x, head, tail0, tail1):
    b, l = x.shape
    flat = x.reshape(-1)
    out = _make_sc_lookup(flat.shape[0])(flat, head, tail0, tail1)
    return out.reshape(b, l, D_MODEL)


# speedup vs baseline: 1.0125x; 1.0125x over previous
import functools

import jax
import jax.numpy as jnp
from jax import lax
from jax.experimental import pallas as pl
from jax.experimental.pallas import tpu as pltpu
from jax.experimental.pallas import tpu_sc as plsc

D_MODEL = 64
CUT0, CUT1, CUT2 = 20000, 200000, 1000000

NC, NS, L = 2, 16, 16
NW = NC * NS
G = 128


def _make_sc_lookup(n):
    per_w = n // NW
    assert n % NW == 0 and per_w % L == 0
    steps = per_w // L
    cap = per_w + G + L

    mesh = plsc.VectorSubcoreMesh(core_axis_name="c", subcore_axis_name="s")

    @functools.partial(
        pl.kernel,
        mesh=mesh,
        compiler_params=pltpu.CompilerParams(
            use_tc_tiling_on_sc=False, needs_layout_passes=False),
        out_type=jax.ShapeDtypeStruct((n, D_MODEL), jnp.float32),
        scratch_types=[
            pltpu.VMEM((per_w,), jnp.int32),
            pltpu.VMEM((cap,), jnp.int32),
            pltpu.VMEM((cap,), jnp.int32),
            pltpu.VMEM((cap,), jnp.int32),
            pltpu.VMEM((cap,), jnp.int32),
            pltpu.VMEM((cap,), jnp.int32),
            pltpu.VMEM((cap,), jnp.int32),
            pltpu.VMEM((G,), jnp.int32),
            pltpu.VMEM((G,), jnp.int32),
            pltpu.VMEM((G, D_MODEL), jnp.float32),
            pltpu.VMEM((G, D_MODEL), jnp.float32),
            pltpu.SemaphoreType.DMA,
            pltpu.SemaphoreType.DMA,
        ],
    )
    def sc_lookup(x_hbm, head_hbm, tail0_hbm, tail1_hbm, out_hbm,
                  idx_v, il0, pl0_, il1, pl1_, il2, pl2_,
                  ps0, ps1, rows0, rows1, sem_g, sem_s):
        wid = lax.axis_index("s") * NC + lax.axis_index("c")
        base = wid * per_w
        pltpu.sync_copy(x_hbm.at[pl.ds(base, per_w)], idx_v)

        lanes = lax.iota(jnp.int32, L)

        dump = jnp.int32(cap - 1)

        def compact(g, offs):
            o0, o1, o2 = offs
            v = idx_v[pl.ds(g * L, L)]
            pos = base + g * L + lanes
            m0 = v < CUT0
            m2 = v >= CUT1
            m1 = jnp.logical_not(m0) & jnp.logical_not(m2)
            new_offs = []
            for m, il, plref, adj, o in (
                (m0, il0, pl0_, v, o0),
                (m1, il1, pl1_, v - CUT0, o1),
                (m2, il2, pl2_, v - CUT1, o2),
            ):
                mi = m.astype(jnp.int32)
                pc = plsc.cumsum(mi)
                dest = jnp.where(m, o + pc - 1, dump)
                plsc.store_scatter(il, [dest], adj)
                plsc.store_scatter(plref, [dest], pos)
                new_offs.append(o + pc[L - 1])
            return tuple(new_offs)

        def compact2(h, offs):
            offs = compact(2 * h, offs)
            return compact(2 * h + 1, offs)

        zero = jnp.int32(0)
        c0, c1, c2 = lax.fori_loop(0, steps // 2, compact2, (zero, zero, zero))

        for table, il, plist, cnt in (
            (head_hbm, il0, pl0_, c0),
            (tail0_hbm, il1, pl1_, c1),
            (tail1_hbm, il2, pl2_, c2),
        ):
            @pl.when(cnt % G != 0)
            def _pad(il=il, plist=plist, cnt=cnt):
                last = jnp.full((L,), cnt - 1, jnp.int32)
                lastv = plsc.load_gather(il, [last])
                lastp = plsc.load_gather(plist, [last])
                for k in range(G // L):
                    il[pl.ds(cnt + k * L, L)] = lastv
                    plist[pl.ds(cnt + k * L, L)] = lastp

            nsub = (cnt + (G - 1)) // G
            rows_bufs = (rows0, rows1)
            ps_bufs = (ps0, ps1)

            def issue_gather(j, parity, table=table, il=il):
                for p in range(2):
                    @pl.when(parity == p)
                    def _():
                        pltpu.async_copy(
                            table.at[il.at[pl.ds(j * G, G)]], rows_bufs[p],
                            sem_g)

            def wait_gather(table=table):
                pltpu.make_async_copy(
                    table.at[il0.at[pl.ds(0, G)]], rows0, sem_g).wait()

            def wait_scatter():
                pltpu.make_async_copy(rows0, out_hbm.at[ps0], sem_s).wait()

            @pl.when(nsub > 0)
            def _(table=table, il=il, plist=plist, nsub=nsub):
                issue_gather(0, 0)

                def gbody(j, _, table=table, il=il, plist=plist, nsub=nsub):
                    parity = j % 2

                    @pl.when(j >= 1)
                    def _():
                        wait_scatter()

                    wait_gather()
                    for p in range(2):
                        @pl.when(parity == p)
                        def _(p=p):
                            for k in range(G // L):
                                ps_bufs[p][pl.ds(k * L, L)] = (
                                    plist[pl.ds(j * G + k * L, L)])
                            pltpu.async_copy(
                                rows_bufs[p], out_hbm.at[ps_bufs[p]], sem_s)

                    @pl.when(j + 1 < nsub)
                    def _():
                        issue_gather(j + 1, (j + 1) % 2)
                    return 0

                lax.fori_loop(0, nsub, gbody, 0)
                wait_scatter()

    return sc_lookup


def kernel(x, head, tail0, tail1):
    b, l = x.shape
    flat = x.reshape(-1)
    out = _make_sc_lookup(flat.shape[0])(flat, head, tail0, tail1)
    return out.reshape(b, l, D_MODEL)
